# Initial kernel scaffold; baseline (speedup 1.0000x reference)
import functools

import jax
import jax.numpy as jnp
from jax import lax
from jax.experimental import pallas as pl
from jax.experimental.pallas import tpu as pltpu
from jax.experimental.pallas import tpu_sc as plsc

N_NODES = 10000
D = 128
N_EDGES = 320000
NT = 16
B = 100
EROWS = N_EDGES // B
RPT = EROWS // NT
PAIRS = RPT // 2
ROWS_PT = N_NODES // NT
DEGW = 16

_mesh = plsc.VectorSubcoreMesh(core_axis_name="c", subcore_axis_name="s")

_deg_struct = jax.ShapeDtypeStruct((N_NODES, DEGW), jnp.float32)
_agg_struct = jax.ShapeDtypeStruct((N_NODES, D), jnp.float32)


@functools.partial(
    pl.kernel,
    out_type=(_deg_struct, _deg_struct, _deg_struct, _deg_struct),
    mesh=_mesh,
    scratch_types=(
        pltpu.VMEM_SHARED((N_NODES, DEGW), jnp.float32),
        pltpu.VMEM_SHARED((N_NODES, DEGW), jnp.float32),
        pltpu.VMEM((RPT, B), jnp.int32),
        pltpu.VMEM((RPT, B), jnp.int32),
        pltpu.VMEM((B, DEGW), jnp.float32),
    ),
)
def _sc_degrees(src0, dst0, src1, dst1, ones_hbm, zeros_hbm,
                outdeg0, indeg0, outdeg1, indeg1,
                deg_out_sh, deg_in_sh, src_v, dst_v, ones_v):
    c = lax.axis_index("c")
    s = lax.axis_index("s")
    sl = pl.ds(s * ROWS_PT, ROWS_PT)
    pltpu.sync_copy(zeros_hbm, deg_out_sh.at[sl])
    pltpu.sync_copy(zeros_hbm, deg_in_sh.at[sl])
    pltpu.sync_copy(ones_hbm, ones_v)
    esl = pl.ds(s * RPT, RPT)

    @pl.when(c == 0)
    def _():
        pltpu.sync_copy(src0.at[esl], src_v)
        pltpu.sync_copy(dst0.at[esl], dst_v)

    @pl.when(c == 1)
    def _():
        pltpu.sync_copy(src1.at[esl], src_v)
        pltpu.sync_copy(dst1.at[esl], dst_v)

    plsc.subcore_barrier()

    @pl.loop(0, RPT)
    def _(j):
        pltpu.sync_copy(ones_v, deg_out_sh.at[src_v.at[j]], add=True)
        pltpu.sync_copy(ones_v, deg_in_sh.at[dst_v.at[j]], add=True)

    plsc.subcore_barrier()

    @pl.when(c == 0)
    def _():
        pltpu.sync_copy(deg_out_sh.at[sl], outdeg0.at[sl])
        pltpu.sync_copy(deg_in_sh.at[sl], indeg0.at[sl])

    @pl.when(c == 1)
    def _():
        pltpu.sync_copy(deg_out_sh.at[sl], outdeg1.at[sl])
        pltpu.sync_copy(deg_in_sh.at[sl], indeg1.at[sl])


@functools.partial(
    pl.kernel,
    out_type=(_agg_struct, _agg_struct),
    mesh=_mesh,
    scratch_types=(
        pltpu.VMEM_SHARED((N_NODES, D), jnp.float32),
        pltpu.VMEM((RPT, B), jnp.int32),
        pltpu.VMEM((RPT, B), jnp.int32),
        pltpu.VMEM((B, D), jnp.float32),
        pltpu.VMEM((B, D), jnp.float32),
        pltpu.SemaphoreType.DMA,
        pltpu.SemaphoreType.DMA,
    ),
)
def _sc_scatter(y0, y1, src0, dst0, src1, dst1, zeros_hbm,
                agg0, agg1,
                agg_sh, src_v, dst_v, rows0, rows1, sem0, sem1):
    c = lax.axis_index("c")
    s = lax.axis_index("s")
    sl = pl.ds(s * ROWS_PT, ROWS_PT)
    pltpu.sync_copy(zeros_hbm, agg_sh.at[sl])
    esl = pl.ds(s * RPT, RPT)

    @pl.when(c == 0)
    def _():
        pltpu.sync_copy(src0.at[esl], src_v)
        pltpu.sync_copy(dst0.at[esl], dst_v)

    @pl.when(c == 1)
    def _():
        pltpu.sync_copy(src1.at[esl], src_v)
        pltpu.sync_copy(dst1.at[esl], dst_v)

    plsc.subcore_barrier()

    def edge_loop(y_hbm):
        def gather(j, buf, sem):
            pltpu.async_copy(y_hbm.at[src_v.at[j]], buf, sem)

        def wait(buf, sem):
            pltpu.make_async_copy(y_hbm.at[src_v.at[0]], buf, sem).wait()

        def scat(j, buf):
            pltpu.sync_copy(buf, agg_sh.at[dst_v.at[j]], add=True)

        gather(0, rows0, sem0)

        @pl.loop(0, PAIRS)
        def _(i):
            j0 = 2 * i
            gather(j0 + 1, rows1, sem1)
            wait(rows0, sem0)
            scat(j0, rows0)

            @pl.when(i < PAIRS - 1)
            def _():
                gather(j0 + 2, rows0, sem0)

            wait(rows1, sem1)
            scat(j0 + 1, rows1)

    @pl.when(c == 0)
    def _():
        edge_loop(y0)

    @pl.when(c == 1)
    def _():
        edge_loop(y1)

    plsc.subcore_barrier()

    @pl.when(c == 0)
    def _():
        pltpu.sync_copy(agg_sh.at[sl], agg0.at[sl])

    @pl.when(c == 1)
    def _():
        pltpu.sync_copy(agg_sh.at[sl], agg1.at[sl])


_RB = 1000


def _y_body(x_ref, d0_ref, d1_ref, w0_ref, w1_ref, y0_ref, y1_ref):
    c0 = lax.rsqrt(jnp.maximum(d0_ref[:, 0:1], 1.0))
    c1 = lax.rsqrt(jnp.maximum(d1_ref[:, 0:1], 1.0))
    xb = x_ref[...]
    y0_ref[...] = jnp.dot(xb * c0, w0_ref[...], preferred_element_type=jnp.float32)
    y1_ref[...] = jnp.dot(xb * c1, w1_ref[...], preferred_element_type=jnp.float32)


def _tc_prepare_y(x, d0, d1, W0, W1):
    return pl.pallas_call(
        _y_body,
        grid=(N_NODES // _RB,),
        in_specs=[
            pl.BlockSpec((_RB, D), lambda i: (i, 0)),
            pl.BlockSpec((_RB, DEGW), lambda i: (i, 0)),
            pl.BlockSpec((_RB, DEGW), lambda i: (i, 0)),
            pl.BlockSpec((D, D), lambda i: (0, 0)),
            pl.BlockSpec((D, D), lambda i: (0, 0)),
        ],
        out_specs=[
            pl.BlockSpec((_RB, D), lambda i: (i, 0)),
            pl.BlockSpec((_RB, D), lambda i: (i, 0)),
        ],
        out_shape=[
            jax.ShapeDtypeStruct((N_NODES, D), jnp.float32),
            jax.ShapeDtypeStruct((N_NODES, D), jnp.float32),
        ],
    )(x, d0, d1, W0, W1)


def _fin_body(a0_ref, a1_ref, d0_ref, d1_ref, b0_ref, b1_ref, o_ref):
    s0 = lax.rsqrt(jnp.maximum(d0_ref[:, 0:1], 1.0))
    s1 = lax.rsqrt(jnp.maximum(d1_ref[:, 0:1], 1.0))
    o_ref[...] = a0_ref[...] * s0 + a1_ref[...] * s1 + b0_ref[...] + b1_ref[...]


def _tc_finalize(agg0, agg1, d0, d1, b0, b1):
    return pl.pallas_call(
        _fin_body,
        grid=(N_NODES // _RB,),
        in_specs=[
            pl.BlockSpec((_RB, D), lambda i: (i, 0)),
            pl.BlockSpec((_RB, D), lambda i: (i, 0)),
            pl.BlockSpec((_RB, DEGW), lambda i: (i, 0)),
            pl.BlockSpec((_RB, DEGW), lambda i: (i, 0)),
            pl.BlockSpec((1, D), lambda i: (0, 0)),
            pl.BlockSpec((1, D), lambda i: (0, 0)),
        ],
        out_specs=pl.BlockSpec((_RB, D), lambda i: (i, 0)),
        out_shape=jax.ShapeDtypeStruct((N_NODES, D), jnp.float32),
    )(agg0, agg1, d0, d1, b0, b1)


def kernel(x, edge_index_rel0, edge_index_rel1, W0, b0, W1, b1):
    src0 = edge_index_rel0[0].astype(jnp.int32).reshape(EROWS, B)
    dst0 = edge_index_rel0[1].astype(jnp.int32).reshape(EROWS, B)
    src1 = edge_index_rel1[0].astype(jnp.int32).reshape(EROWS, B)
    dst1 = edge_index_rel1[1].astype(jnp.int32).reshape(EROWS, B)
    ones_hbm = jnp.ones((B, DEGW), jnp.float32)
    zeros_deg = jnp.zeros((ROWS_PT, DEGW), jnp.float32)
    zeros_agg = jnp.zeros((ROWS_PT, D), jnp.float32)

    outdeg0, indeg0, outdeg1, indeg1 = _sc_degrees(
        src0, dst0, src1, dst1, ones_hbm, zeros_deg)
    y0, y1 = _tc_prepare_y(x, outdeg0, outdeg1, W0, W1)
    agg0, agg1 = _sc_scatter(y0, y1, src0, dst0, src1, dst1, zeros_agg)
    return _tc_finalize(agg0, agg1, indeg0, indeg1,
                        b0.reshape(1, D), b1.reshape(1, D))


# speedup vs baseline: 9.8058x; 9.8058x over previous
import functools

import jax
import jax.numpy as jnp
from jax import lax
from jax.experimental import pallas as pl
from jax.experimental.pallas import tpu as pltpu
from jax.experimental.pallas import tpu_sc as plsc

N_NODES = 10000
D = 128
N_EDGES = 320000
NT = 16
B = 100
EROWS = N_EDGES // B
RPT = EROWS // NT
PAIRS = RPT // 2
CH = 20
NCH = RPT // CH
ROWS_PT = 624
TAIL_BASE = ROWS_PT * NT
TAIL = N_NODES - TAIL_BASE
DEGW = 16

_deg_struct = jax.ShapeDtypeStruct((N_NODES, DEGW), jnp.float32)
_agg_struct = jax.ShapeDtypeStruct((N_NODES, D), jnp.float32)


@functools.cache
def _sc_kernels():
    mesh = plsc.VectorSubcoreMesh(core_axis_name="c", subcore_axis_name="s")
    params = pltpu.CompilerParams(use_tc_tiling_on_sc=False)

    @functools.partial(
        pl.kernel,
        out_type=(_deg_struct, _deg_struct, _deg_struct, _deg_struct),
        mesh=mesh,
        compiler_params=params,
        scratch_types=(
            pltpu.VMEM_SHARED((N_NODES, DEGW), jnp.float32),
            pltpu.VMEM_SHARED((N_NODES, DEGW), jnp.float32),
            pltpu.VMEM((RPT, B), jnp.int32),
            pltpu.VMEM((RPT, B), jnp.int32),
            pltpu.VMEM((B, DEGW), jnp.float32),
        ),
    )
    def sc_degrees(src0, dst0, src1, dst1, ones_hbm, zeros_hbm,
                   outdeg0, indeg0, outdeg1, indeg1,
                   deg_out_sh, deg_in_sh, src_v, dst_v, ones_v):
        c = lax.axis_index("c")
        s = lax.axis_index("s")
        sl = pl.ds(s * ROWS_PT, ROWS_PT)
        tsl = pl.ds(TAIL_BASE, TAIL)
        pltpu.sync_copy(zeros_hbm.at[pl.ds(0, ROWS_PT)], deg_out_sh.at[sl])
        pltpu.sync_copy(zeros_hbm.at[pl.ds(0, ROWS_PT)], deg_in_sh.at[sl])

        @pl.when(s == NT - 1)
        def _():
            pltpu.sync_copy(zeros_hbm.at[pl.ds(0, TAIL)], deg_out_sh.at[tsl])
            pltpu.sync_copy(zeros_hbm.at[pl.ds(0, TAIL)], deg_in_sh.at[tsl])

        pltpu.sync_copy(ones_hbm, ones_v)
        esl = pl.ds(s * RPT, RPT)

        @pl.when(c == 0)
        def _():
            pltpu.sync_copy(src0.at[esl], src_v)
            pltpu.sync_copy(dst0.at[esl], dst_v)

        @pl.when(c == 1)
        def _():
            pltpu.sync_copy(src1.at[esl], src_v)
            pltpu.sync_copy(dst1.at[esl], dst_v)

        plsc.subcore_barrier()

        @pl.loop(0, RPT)
        def _(j):
            pltpu.sync_copy(ones_v, deg_out_sh.at[src_v.at[j]], add=True)
            pltpu.sync_copy(ones_v, deg_in_sh.at[dst_v.at[j]], add=True)

        plsc.subcore_barrier()

        @pl.when(c == 0)
        def _():
            pltpu.sync_copy(deg_out_sh.at[sl], outdeg0.at[sl])
            pltpu.sync_copy(deg_in_sh.at[sl], indeg0.at[sl])

            @pl.when(s == NT - 1)
            def _():
                pltpu.sync_copy(deg_out_sh.at[tsl], outdeg0.at[tsl])
                pltpu.sync_copy(deg_in_sh.at[tsl], indeg0.at[tsl])

        @pl.when(c == 1)
        def _():
            pltpu.sync_copy(deg_out_sh.at[sl], outdeg1.at[sl])
            pltpu.sync_copy(deg_in_sh.at[sl], indeg1.at[sl])

            @pl.when(s == NT - 1)
            def _():
                pltpu.sync_copy(deg_out_sh.at[tsl], outdeg1.at[tsl])
                pltpu.sync_copy(deg_in_sh.at[tsl], indeg1.at[tsl])

    @functools.partial(
        pl.kernel,
        out_type=(_agg_struct, _agg_struct),
        mesh=mesh,
        compiler_params=params,
        scratch_types=(
            pltpu.VMEM_SHARED((N_NODES, D), jnp.float32),
            pltpu.VMEM((CH, B), jnp.int32),
            pltpu.VMEM((CH, B), jnp.int32),
            pltpu.VMEM((B, D), jnp.float32),
            pltpu.VMEM((B, D), jnp.float32),
            pltpu.SemaphoreType.DMA,
            pltpu.SemaphoreType.DMA,
        ),
    )
    def sc_scatter(y0, y1, src0, dst0, src1, dst1, zeros_hbm,
                   agg0, agg1,
                   agg_sh, src_v, dst_v, rows0, rows1, sem0, sem1):
        c = lax.axis_index("c")
        s = lax.axis_index("s")
        sl = pl.ds(s * ROWS_PT, ROWS_PT)
        tsl = pl.ds(TAIL_BASE, TAIL)
        pltpu.sync_copy(zeros_hbm.at[pl.ds(0, ROWS_PT)], agg_sh.at[sl])

        @pl.when(s == NT - 1)
        def _():
            pltpu.sync_copy(zeros_hbm.at[pl.ds(0, TAIL)], agg_sh.at[tsl])

        plsc.subcore_barrier()

        erow0 = s * RPT

        def edge_loop(y_hbm, src_hbm, dst_hbm):
            def gather(j, buf, sem):
                pltpu.async_copy(y_hbm.at[src_v.at[j]], buf, sem)

            def wait(buf, sem):
                pltpu.make_async_copy(y_hbm.at[src_v.at[0]], buf, sem).wait()

            def scat(j, buf):
                pltpu.sync_copy(buf, agg_sh.at[dst_v.at[j]], add=True)

            @pl.loop(0, NCH)
            def _(k):
                csl = pl.ds(erow0 + k * CH, CH)
                pltpu.sync_copy(src_hbm.at[csl], src_v)
                pltpu.sync_copy(dst_hbm.at[csl], dst_v)

                gather(0, rows0, sem0)

                @pl.loop(0, CH // 2)
                def _(i):
                    j0 = 2 * i
                    gather(j0 + 1, rows1, sem1)
                    wait(rows0, sem0)
                    scat(j0, rows0)

                    @pl.when(i < CH // 2 - 1)
                    def _():
                        gather(j0 + 2, rows0, sem0)

                    wait(rows1, sem1)
                    scat(j0 + 1, rows1)

        @pl.when(c == 0)
        def _():
            edge_loop(y0, src0, dst0)

        @pl.when(c == 1)
        def _():
            edge_loop(y1, src1, dst1)

        plsc.subcore_barrier()

        @pl.when(c == 0)
        def _():
            pltpu.sync_copy(agg_sh.at[sl], agg0.at[sl])

            @pl.when(s == NT - 1)
            def _():
                pltpu.sync_copy(agg_sh.at[tsl], agg0.at[tsl])

        @pl.when(c == 1)
        def _():
            pltpu.sync_copy(agg_sh.at[sl], agg1.at[sl])

            @pl.when(s == NT - 1)
            def _():
                pltpu.sync_copy(agg_sh.at[tsl], agg1.at[tsl])

    return sc_degrees, sc_scatter


_RB = 1000


def _y_body(x_ref, d0_ref, d1_ref, w0_ref, w1_ref, y0_ref, y1_ref):
    c0 = lax.rsqrt(jnp.maximum(d0_ref[:, 0:1], 1.0))
    c1 = lax.rsqrt(jnp.maximum(d1_ref[:, 0:1], 1.0))
    xb = x_ref[...]
    y0_ref[...] = jnp.dot(xb * c0, w0_ref[...], preferred_element_type=jnp.float32)
    y1_ref[...] = jnp.dot(xb * c1, w1_ref[...], preferred_element_type=jnp.float32)


def _tc_prepare_y(x, d0, d1, W0, W1):
    return pl.pallas_call(
        _y_body,
        grid=(N_NODES // _RB,),
        in_specs=[
            pl.BlockSpec((_RB, D), lambda i: (i, 0)),
            pl.BlockSpec((_RB, DEGW), lambda i: (i, 0)),
            pl.BlockSpec((_RB, DEGW), lambda i: (i, 0)),
            pl.BlockSpec((D, D), lambda i: (0, 0)),
            pl.BlockSpec((D, D), lambda i: (0, 0)),
        ],
        out_specs=[
            pl.BlockSpec((_RB, D), lambda i: (i, 0)),
            pl.BlockSpec((_RB, D), lambda i: (i, 0)),
        ],
        out_shape=[
            jax.ShapeDtypeStruct((N_NODES, D), jnp.float32),
            jax.ShapeDtypeStruct((N_NODES, D), jnp.float32),
        ],
    )(x, d0, d1, W0, W1)


def _fin_body(a0_ref, a1_ref, d0_ref, d1_ref, b0_ref, b1_ref, o_ref):
    s0 = lax.rsqrt(jnp.maximum(d0_ref[:, 0:1], 1.0))
    s1 = lax.rsqrt(jnp.maximum(d1_ref[:, 0:1], 1.0))
    o_ref[...] = a0_ref[...] * s0 + a1_ref[...] * s1 + b0_ref[...] + b1_ref[...]


def _tc_finalize(agg0, agg1, d0, d1, b0, b1):
    return pl.pallas_call(
        _fin_body,
        grid=(N_NODES // _RB,),
        in_specs=[
            pl.BlockSpec((_RB, D), lambda i: (i, 0)),
            pl.BlockSpec((_RB, D), lambda i: (i, 0)),
            pl.BlockSpec((_RB, DEGW), lambda i: (i, 0)),
            pl.BlockSpec((_RB, DEGW), lambda i: (i, 0)),
            pl.BlockSpec((1, D), lambda i: (0, 0)),
            pl.BlockSpec((1, D), lambda i: (0, 0)),
        ],
        out_specs=pl.BlockSpec((_RB, D), lambda i: (i, 0)),
        out_shape=jax.ShapeDtypeStruct((N_NODES, D), jnp.float32),
    )(agg0, agg1, d0, d1, b0, b1)


def kernel(x, edge_index_rel0, edge_index_rel1, W0, b0, W1, b1):
    src0 = edge_index_rel0[0].astype(jnp.int32).reshape(EROWS, B)
    dst0 = edge_index_rel0[1].astype(jnp.int32).reshape(EROWS, B)
    src1 = edge_index_rel1[0].astype(jnp.int32).reshape(EROWS, B)
    dst1 = edge_index_rel1[1].astype(jnp.int32).reshape(EROWS, B)
    ones_hbm = jnp.ones((B, DEGW), jnp.float32)
    zeros_deg = jnp.zeros((ROWS_PT, DEGW), jnp.float32)
    zeros_agg = jnp.zeros((ROWS_PT, D), jnp.float32)

    sc_degrees, sc_scatter = _sc_kernels()
    outdeg0, indeg0, outdeg1, indeg1 = sc_degrees(
        src0, dst0, src1, dst1, ones_hbm, zeros_deg)
    y0, y1 = _tc_prepare_y(x, outdeg0, outdeg1, W0, W1)
    agg0, agg1 = sc_scatter(y0, y1, src0, dst0, src1, dst1, zeros_agg)
    return _tc_finalize(agg0, agg1, indeg0, indeg1,
                        b0.reshape(1, D), b1.reshape(1, D))
